# baseline (speedup 1.0000x reference)
import jax
import jax.numpy as jnp
from jax import lax
from jax.experimental import pallas as pl
from jax.experimental.pallas import tpu as pltpu
from jax.experimental.pallas import tpu_sc as plsc

N_SEG = 10000
D = 128
N_ROWS = 320000
NC, NS = 2, 16
NW = NC * NS

CHUNK = 128
SC_ROWS = 147456
BASE = SC_ROWS // CHUNK // NW
N_SEG_PAD = 10240
SEG_PER_TILE = N_SEG_PAD // NS

B = 256
W = 256
TC_BLK0 = SC_ROWS // B
TC_NBLK = (N_ROWS - SC_ROWS) // B
N_SEG_TC = N_SEG + W + 8


def _sc_body(x_hbm, idx_hbm, zeros_hbm, out_hbm,
             rows0, rows1, idx0, idx1, acc, sem0, sem1):
    c = lax.axis_index("c")
    s = lax.axis_index("s")
    wid = c * NS + s

    pltpu.sync_copy(zeros_hbm.at[pl.ds(s * SEG_PER_TILE, SEG_PER_TILE), :],
                    acc.at[pl.ds(s * SEG_PER_TILE, SEG_PER_TILE), :])
    plsc.subcore_barrier()

    start = wid * BASE
    bufs = ((rows0, idx0, sem0), (rows1, idx1, sem1))

    def fire(j, b):
        rows, idx, sem = bufs[b]
        pltpu.async_copy(x_hbm.at[pl.ds((start + j) * CHUNK, CHUNK), :], rows, sem)
        pltpu.async_copy(idx_hbm.at[pl.ds((start + j) * CHUNK, CHUNK)], idx, sem)

    def drain_and_scatter(b):
        rows, idx, sem = bufs[b]
        pltpu.make_async_copy(x_hbm.at[pl.ds(0, CHUNK), :], rows, sem).wait()
        pltpu.make_async_copy(idx_hbm.at[pl.ds(0, CHUNK)], idx, sem).wait()
        pltpu.sync_copy(rows, acc.at[idx], add=True)

    fire(0, 0)

    def body(j, carry):
        for b in range(2):
            parity = jnp.equal(lax.rem(j, 2), b)

            @pl.when(parity & (j + 1 < BASE))
            def _():
                fire(j + 1, 1 - b)

            @pl.when(parity)
            def _():
                drain_and_scatter(b)
        return carry

    lax.fori_loop(0, BASE, body, 0)
    plsc.subcore_barrier()

    pltpu.sync_copy(acc.at[pl.ds(s * SEG_PER_TILE, SEG_PER_TILE), :],
                    out_hbm.at[c, pl.ds(s * SEG_PER_TILE, SEG_PER_TILE), :])


def _tc_pool(idx_s, x_ref, idx_v, o_ref):
    i = pl.program_id(0)

    @pl.when(i == 0)
    def _():
        o_ref[...] = jnp.zeros_like(o_ref)

    first = idx_s[0, 0, 0]
    last = idx_s[0, 0, B - 1]
    base0 = (first // 8) * 8
    n_win = (last - base0) // W + 1

    xb = x_ref[...]
    xhi = xb.astype(jnp.bfloat16)
    xlo = (xb - xhi.astype(jnp.float32)).astype(jnp.bfloat16)
    idx_row = idx_v[0, 0, :].reshape(1, B)

    def win_body(w, carry):
        base = pl.multiple_of(base0 + w * W, 8)
        seg = base + lax.broadcasted_iota(jnp.int32, (W, B), 0)
        oneh = (seg == idx_row).astype(jnp.bfloat16)
        part = (jax.lax.dot(oneh, xhi, preferred_element_type=jnp.float32)
                + jax.lax.dot(oneh, xlo, preferred_element_type=jnp.float32))
        o_ref[pl.ds(base, W), :] += part
        return carry

    lax.fori_loop(0, n_win, win_body, 0)


def _tc_combine(p_ref, t_ref, o_ref):
    o_ref[...] = p_ref[0] + p_ref[1] + t_ref[...]


def kernel(x, index):
    idx32 = index.astype(jnp.int32)
    zeros = jnp.zeros((N_SEG_PAD, D), dtype=jnp.float32)

    mesh = plsc.VectorSubcoreMesh(core_axis_name="c", subcore_axis_name="s")
    partials = pl.kernel(
        _sc_body,
        out_type=jax.ShapeDtypeStruct((NC, N_SEG_PAD, D), jnp.float32),
        mesh=mesh,
        scratch_types=[
            pltpu.VMEM((CHUNK, D), jnp.float32),
            pltpu.VMEM((CHUNK, D), jnp.float32),
            pltpu.VMEM((CHUNK,), jnp.int32),
            pltpu.VMEM((CHUNK,), jnp.int32),
            pltpu.VMEM_SHARED((N_SEG_PAD, D), jnp.float32),
            pltpu.SemaphoreType.DMA,
            pltpu.SemaphoreType.DMA,
        ],
    )(x, idx32, zeros)

    idx3 = idx32.reshape(N_ROWS // B, 1, B)
    tc_part = pl.pallas_call(
        _tc_pool,
        grid=(TC_NBLK,),
        in_specs=[
            pl.BlockSpec((1, 1, B), lambda i: (i + TC_BLK0, 0, 0),
                         memory_space=pltpu.SMEM),
            pl.BlockSpec((B, D), lambda i: (i + TC_BLK0, 0)),
            pl.BlockSpec((1, 1, B), lambda i: (i + TC_BLK0, 0, 0)),
        ],
        out_specs=pl.BlockSpec((N_SEG_TC, D), lambda i: (0, 0)),
        out_shape=jax.ShapeDtypeStruct((N_SEG_TC, D), jnp.float32),
    )(idx3, x, idx3)

    blk = 1000
    out = pl.pallas_call(
        _tc_combine,
        grid=(N_SEG // blk,),
        in_specs=[
            pl.BlockSpec((NC, blk, D), lambda i: (0, i, 0)),
            pl.BlockSpec((blk, D), lambda i: (i, 0)),
        ],
        out_specs=pl.BlockSpec((blk, D), lambda i: (i, 0)),
        out_shape=jax.ShapeDtypeStruct((N_SEG, D), jnp.float32),
    )(partials, tc_part)
    return out


# speedup vs baseline: 1.7522x; 1.7522x over previous
import jax
import jax.numpy as jnp
from jax import lax
from jax.experimental import pallas as pl
from jax.experimental.pallas import tpu as pltpu
from jax.experimental.pallas import tpu_sc as plsc

N_SEG = 10000
D = 128
N_ROWS = 320000
NC, NS = 2, 16
NW = NC * NS

CHUNK = 128
SC_ROWS = 147456
BASE = SC_ROWS // CHUNK // NW
N_SEG_PAD = 10240
SEG_PER_TILE = N_SEG_PAD // NS

B = 512
W = 64
TC_BLK0 = SC_ROWS // B
TC_NBLK = (N_ROWS - SC_ROWS) // B
N_SEG_TC = N_SEG + W + 16


def _sc_body(x_hbm, idx_hbm, zeros_hbm, out_hbm,
             rows0, rows1, idx0, idx1, acc, sem0, sem1):
    c = lax.axis_index("c")
    s = lax.axis_index("s")
    wid = c * NS + s

    pltpu.sync_copy(zeros_hbm.at[pl.ds(s * SEG_PER_TILE, SEG_PER_TILE), :],
                    acc.at[pl.ds(s * SEG_PER_TILE, SEG_PER_TILE), :])
    plsc.subcore_barrier()

    start = wid * BASE
    bufs = ((rows0, idx0, sem0), (rows1, idx1, sem1))

    def fire(j, b):
        rows, idx, sem = bufs[b]
        pltpu.async_copy(x_hbm.at[pl.ds((start + j) * CHUNK, CHUNK), :], rows, sem)
        pltpu.async_copy(idx_hbm.at[pl.ds((start + j) * CHUNK, CHUNK)], idx, sem)

    def drain_and_scatter(b):
        rows, idx, sem = bufs[b]
        pltpu.make_async_copy(x_hbm.at[pl.ds(0, CHUNK), :], rows, sem).wait()
        pltpu.make_async_copy(idx_hbm.at[pl.ds(0, CHUNK)], idx, sem).wait()
        pltpu.sync_copy(rows, acc.at[idx], add=True)

    fire(0, 0)

    def body(j, carry):
        for b in range(2):
            parity = jnp.equal(lax.rem(j, 2), b)

            @pl.when(parity & (j + 1 < BASE))
            def _():
                fire(j + 1, 1 - b)

            @pl.when(parity)
            def _():
                drain_and_scatter(b)
        return carry

    lax.fori_loop(0, BASE, body, 0)
    plsc.subcore_barrier()

    pltpu.sync_copy(acc.at[pl.ds(s * SEG_PER_TILE, SEG_PER_TILE), :],
                    out_hbm.at[c, pl.ds(s * SEG_PER_TILE, SEG_PER_TILE), :])


def _tc_pool(idx_s, x_ref, idx_v, o_ref):
    i = pl.program_id(0)

    @pl.when(i == 0)
    def _():
        o_ref[...] = jnp.zeros_like(o_ref)

    first = idx_s[0, 0, 0]
    last = idx_s[0, 0, B - 1]
    base0 = (first // 8) * 8
    n_win = (last - base0) // W + 1

    xb = x_ref[...]
    xhi = xb.astype(jnp.bfloat16)
    xlo = (xb - xhi.astype(jnp.float32)).astype(jnp.bfloat16)
    idx_row = idx_v[0, 0, :].reshape(1, B)

    def win_body(w, carry):
        base = pl.multiple_of(base0 + w * W, 8)
        seg = base + lax.broadcasted_iota(jnp.int32, (W, B), 0)
        oneh = (seg == idx_row).astype(jnp.bfloat16)
        part = (jax.lax.dot(oneh, xhi, preferred_element_type=jnp.float32)
                + jax.lax.dot(oneh, xlo, preferred_element_type=jnp.float32))
        o_ref[pl.ds(base, W), :] += part
        return carry

    lax.fori_loop(0, n_win, win_body, 0)


def _tc_combine(p_ref, t_ref, o_ref):
    o_ref[...] = p_ref[0] + p_ref[1] + t_ref[...]


def kernel(x, index):
    idx32 = index.astype(jnp.int32)
    zeros = jnp.zeros((N_SEG_PAD, D), dtype=jnp.float32)

    mesh = plsc.VectorSubcoreMesh(core_axis_name="c", subcore_axis_name="s")
    partials = pl.kernel(
        _sc_body,
        out_type=jax.ShapeDtypeStruct((NC, N_SEG_PAD, D), jnp.float32),
        mesh=mesh,
        scratch_types=[
            pltpu.VMEM((CHUNK, D), jnp.float32),
            pltpu.VMEM((CHUNK, D), jnp.float32),
            pltpu.VMEM((CHUNK,), jnp.int32),
            pltpu.VMEM((CHUNK,), jnp.int32),
            pltpu.VMEM_SHARED((N_SEG_PAD, D), jnp.float32),
            pltpu.SemaphoreType.DMA,
            pltpu.SemaphoreType.DMA,
        ],
    )(x, idx32, zeros)

    idx3 = idx32.reshape(N_ROWS // B, 1, B)
    tc_part = pl.pallas_call(
        _tc_pool,
        grid=(TC_NBLK,),
        in_specs=[
            pl.BlockSpec((1, 1, B), lambda i: (i + TC_BLK0, 0, 0),
                         memory_space=pltpu.SMEM),
            pl.BlockSpec((B, D), lambda i: (i + TC_BLK0, 0)),
            pl.BlockSpec((1, 1, B), lambda i: (i + TC_BLK0, 0, 0)),
        ],
        out_specs=pl.BlockSpec((N_SEG_TC, D), lambda i: (0, 0)),
        out_shape=jax.ShapeDtypeStruct((N_SEG_TC, D), jnp.float32),
    )(idx3, x, idx3)

    blk = 1000
    out = pl.pallas_call(
        _tc_combine,
        grid=(N_SEG // blk,),
        in_specs=[
            pl.BlockSpec((NC, blk, D), lambda i: (0, i, 0)),
            pl.BlockSpec((blk, D), lambda i: (i, 0)),
        ],
        out_specs=pl.BlockSpec((blk, D), lambda i: (i, 0)),
        out_shape=jax.ShapeDtypeStruct((N_SEG, D), jnp.float32),
    )(partials, tc_part)
    return out


# speedup vs baseline: 3.0379x; 1.7338x over previous
import jax
import jax.numpy as jnp
from jax import lax
from jax.experimental import pallas as pl
from jax.experimental.pallas import tpu as pltpu
from jax.experimental.pallas import tpu_sc as plsc

N_SEG = 10000
D = 128
N_ROWS = 320000
CHUNK = 128
NC, NS = 2, 16
NW = NC * NS
TOTAL_CHUNKS = N_ROWS // CHUNK
BASE = TOTAL_CHUNKS // NW
EXTRA = TOTAL_CHUNKS - BASE * NW
N_SEG_PAD = 10240
SEG_PER_TILE = N_SEG_PAD // NS


def _sc_body(x_hbm, idx_hbm, idx3_hbm, zeros_hbm, out_hbm,
             rows0, rows1, idx_all, idx_tail, acc, sem0, sem1):
    c = lax.axis_index("c")
    s = lax.axis_index("s")
    wid = c * NS + s

    pltpu.sync_copy(zeros_hbm.at[pl.ds(s * SEG_PER_TILE, SEG_PER_TILE), :],
                    acc.at[pl.ds(s * SEG_PER_TILE, SEG_PER_TILE), :])
    plsc.subcore_barrier()

    start = wid * BASE
    bufs = ((rows0, sem0), (rows1, sem1))

    pltpu.sync_copy(idx3_hbm.at[pl.ds(start, BASE)], idx_all)

    def fire(j, b):
        rows, sem = bufs[b]
        pltpu.async_copy(x_hbm.at[pl.ds((start + j) * CHUNK, CHUNK), :], rows, sem)

    def drain_and_scatter(j, b):
        rows, sem = bufs[b]
        pltpu.make_async_copy(x_hbm.at[pl.ds(0, CHUNK), :], rows, sem).wait()
        pltpu.sync_copy(rows, acc.at[idx_all.at[j, 0]], add=True)

    fire(0, 0)

    def body(j, carry):
        for b in range(2):
            parity = jnp.equal(lax.rem(j, 2), b)

            @pl.when(parity & (j + 1 < BASE))
            def _():
                fire(j + 1, 1 - b)

            @pl.when(parity)
            def _():
                drain_and_scatter(j, b)
        return carry

    lax.fori_loop(0, BASE, body, 0)

    @pl.when(wid < EXTRA)
    def _():
        j = NW * BASE + wid
        pltpu.sync_copy(x_hbm.at[pl.ds(j * CHUNK, CHUNK), :], rows0)
        pltpu.sync_copy(idx_hbm.at[pl.ds(j * CHUNK, CHUNK)], idx_tail)
        pltpu.sync_copy(rows0, acc.at[idx_tail], add=True)

    plsc.subcore_barrier()

    pltpu.sync_copy(acc.at[pl.ds(s * SEG_PER_TILE, SEG_PER_TILE), :],
                    out_hbm.at[c, pl.ds(s * SEG_PER_TILE, SEG_PER_TILE), :])


def _tc_add(p_ref, o_ref):
    o_ref[...] = p_ref[0] + p_ref[1]


def kernel(x, index):
    idx32 = index.astype(jnp.int32)
    zeros = jnp.zeros((N_SEG_PAD, D), dtype=jnp.float32)

    mesh = plsc.VectorSubcoreMesh(core_axis_name="c", subcore_axis_name="s")
    partials = pl.kernel(
        _sc_body,
        out_type=jax.ShapeDtypeStruct((NC, N_SEG_PAD, D), jnp.float32),
        mesh=mesh,
        scratch_types=[
            pltpu.VMEM((CHUNK, D), jnp.float32),
            pltpu.VMEM((CHUNK, D), jnp.float32),
            pltpu.VMEM((BASE, 1, CHUNK), jnp.int32),
            pltpu.VMEM((CHUNK,), jnp.int32),
            pltpu.VMEM_SHARED((N_SEG_PAD, D), jnp.float32),
            pltpu.SemaphoreType.DMA,
            pltpu.SemaphoreType.DMA,
        ],
    )(x, idx32, idx32.reshape(TOTAL_CHUNKS, 1, CHUNK), zeros)

    blk = 1000
    out = pl.pallas_call(
        _tc_add,
        grid=(N_SEG // blk,),
        in_specs=[pl.BlockSpec((NC, blk, D), lambda i: (0, i, 0))],
        out_specs=pl.BlockSpec((blk, D), lambda i: (i, 0)),
        out_shape=jax.ShapeDtypeStruct((N_SEG, D), jnp.float32),
    )(partials)
    return out


# speedup vs baseline: 3.3040x; 1.0876x over previous
import jax
import jax.numpy as jnp
from jax import lax
from jax.experimental import pallas as pl
from jax.experimental.pallas import tpu as pltpu
from jax.experimental.pallas import tpu_sc as plsc

N_SEG = 10000
D = 128
N_ROWS = 320000
CHUNK = 128
NC, NS = 2, 16
NW = NC * NS
TOTAL_CHUNKS = N_ROWS // CHUNK
BASE = TOTAL_CHUNKS // NW
EXTRA = TOTAL_CHUNKS - BASE * NW
N_SEG_PAD = 10112
SEG_PER_TILE = N_SEG_PAD // NS


def _sc_body(x_hbm, idx_hbm, zeros_hbm, out_hbm,
             rows0, rows1, rows2, idx0, idx1, idx2, acc, sem0, sem1, sem2):
    c = lax.axis_index("c")
    s = lax.axis_index("s")
    wid = c * NS + s

    pltpu.sync_copy(zeros_hbm.at[pl.ds(s * SEG_PER_TILE, SEG_PER_TILE), :],
                    acc.at[pl.ds(s * SEG_PER_TILE, SEG_PER_TILE), :])
    plsc.subcore_barrier()

    start = wid * BASE
    bufs = ((rows0, idx0, sem0), (rows1, idx1, sem1), (rows2, idx2, sem2))

    def fire(j, b):
        rows, idx, sem = bufs[b]
        pltpu.async_copy(x_hbm.at[pl.ds((start + j) * CHUNK, CHUNK), :], rows, sem)
        pltpu.async_copy(idx_hbm.at[pl.ds((start + j) * CHUNK, CHUNK)], idx, sem)

    def drain_and_scatter(b):
        rows, idx, sem = bufs[b]
        pltpu.make_async_copy(x_hbm.at[pl.ds(0, CHUNK), :], rows, sem).wait()
        pltpu.make_async_copy(idx_hbm.at[pl.ds(0, CHUNK)], idx, sem).wait()
        pltpu.sync_copy(rows, acc.at[idx], add=True)

    fire(0, 0)
    fire(1, 1)

    def body(j, carry):
        for b in range(3):
            parity = jnp.equal(lax.rem(j, 3), b)

            @pl.when(parity & (j + 2 < BASE))
            def _():
                fire(j + 2, (b + 2) % 3)

            @pl.when(parity)
            def _():
                drain_and_scatter(b)
        return carry

    lax.fori_loop(0, BASE, body, 0)

    @pl.when(wid < EXTRA)
    def _():
        j = NW * BASE + wid
        pltpu.sync_copy(x_hbm.at[pl.ds(j * CHUNK, CHUNK), :], rows0)
        pltpu.sync_copy(idx_hbm.at[pl.ds(j * CHUNK, CHUNK)], idx0)
        pltpu.sync_copy(rows0, acc.at[idx0], add=True)

    plsc.subcore_barrier()

    pltpu.sync_copy(acc.at[pl.ds(s * SEG_PER_TILE, SEG_PER_TILE), :],
                    out_hbm.at[c, pl.ds(s * SEG_PER_TILE, SEG_PER_TILE), :])


def _tc_add(p_ref, o_ref):
    o_ref[...] = p_ref[0] + p_ref[1]


def kernel(x, index):
    idx32 = index.astype(jnp.int32)
    zeros = jnp.zeros((N_SEG_PAD, D), dtype=jnp.float32)

    mesh = plsc.VectorSubcoreMesh(core_axis_name="c", subcore_axis_name="s")
    partials = pl.kernel(
        _sc_body,
        out_type=jax.ShapeDtypeStruct((NC, N_SEG_PAD, D), jnp.float32),
        mesh=mesh,
        scratch_types=[
            pltpu.VMEM((CHUNK, D), jnp.float32),
            pltpu.VMEM((CHUNK, D), jnp.float32),
            pltpu.VMEM((CHUNK, D), jnp.float32),
            pltpu.VMEM((CHUNK,), jnp.int32),
            pltpu.VMEM((CHUNK,), jnp.int32),
            pltpu.VMEM((CHUNK,), jnp.int32),
            pltpu.VMEM_SHARED((N_SEG_PAD, D), jnp.float32),
            pltpu.SemaphoreType.DMA,
            pltpu.SemaphoreType.DMA,
            pltpu.SemaphoreType.DMA,
        ],
    )(x, idx32, zeros)

    blk = 1000
    out = pl.pallas_call(
        _tc_add,
        grid=(N_SEG // blk,),
        in_specs=[pl.BlockSpec((NC, blk, D), lambda i: (0, i, 0))],
        out_specs=pl.BlockSpec((blk, D), lambda i: (i, 0)),
        out_shape=jax.ShapeDtypeStruct((N_SEG, D), jnp.float32),
    )(partials)
    return out
